# baseline (speedup 1.0000x reference)
import jax
import jax.numpy as jnp
from jax import lax
from jax.experimental import pallas as pl
from jax.experimental.pallas import tpu as pltpu
from jax.experimental.pallas import tpu_sc as plsc

_BINS = 10
_ROWS, _COLS = 16384, 1000

_SC_ROWS = 0
_TC_ROWS = _ROWS - _SC_ROWS

_NSC = _SC_ROWS * _COLS
_NC, _NS, _L = 2, 16, 16
_NW = _NC * _NS
_PER_W = _NSC // _NW
_CH = 16000
_NCH = _PER_W // _CH
_VECS = _CH // _L
_K = 8

_BM = 512
_G = _TC_ROWS // _BM
_EDGES = [(i + 1) / 10.0 for i in range(_BINS - 1)] + [1.0 + 1e-6]

_P = (6.9445741e-05, 9.9626195e-01, -4.6644244e-01, 2.1866548e-01,
      -5.5459313e-02)


def _log1p_poly(z):
    c0, c1, c2, c3, c4 = (jnp.float32(c) for c in _P)
    z2 = z * z
    return (c0 + c1 * z) + z2 * ((c2 + c3 * z) + c4 * z2)


def _sc_body(pred_hbm, targ_hbm, cnt_out, sum_out,
             pbuf0, pbuf1, tbuf0, tbuf1, cnt_acc, sum_acc, sem0, sem1):
    wid = lax.axis_index("s") * _NC + lax.axis_index("c")
    base = (_ROWS - _SC_ROWS) * _COLS + wid * _PER_W
    sems = (sem0, sem1)
    lane = lax.iota(jnp.int32, _L)
    zeros16 = jnp.zeros((_L,), jnp.float32)

    for v in range(_BINS):
        cnt_acc[pl.ds(v * _L, _L)] = zeros16
        sum_acc[pl.ds(v * _L, _L)] = zeros16

    pbufs = (pbuf0, pbuf1)
    tbufs = (tbuf0, tbuf1)

    def start(c, b):
        src = pl.ds(base + c * _CH, _CH)
        pltpu.make_async_copy(pred_hbm.at[src], pbufs[b], sems[b]).start()
        pltpu.make_async_copy(targ_hbm.at[src], tbufs[b], sems[b]).start()

    def wait(b):
        dummy = pl.ds(0, _CH)
        pltpu.make_async_copy(pred_hbm.at[dummy], pbufs[b], sems[b]).wait()
        pltpu.make_async_copy(targ_hbm.at[dummy], tbufs[b], sems[b]).wait()

    ones16 = jnp.ones((_L,), jnp.float32)

    def compute(b):
        pb = pbufs[b]
        tb = tbufs[b]

        def inner(j, carry):
            base_j = j * (_K * _L)
            ps = [pb[pl.ds(base_j + k * _L, _L)] for k in range(_K)]
            ts = [tb[pl.ds(base_j + k * _L, _L)] for k in range(_K)]
            zs = [jnp.exp(-jnp.abs(p)) for p in ps]
            l1s = [_log1p_poly(z) for z in zs]
            bces = [jnp.maximum(p, 0.0) - p * t + l1
                    for p, t, l1 in zip(ps, ts, l1s)]
            gs = [jnp.abs(p - t) for p, t in zip(ps, ts)]
            bis = [(jnp.minimum(g, 0.95) * 10.0).astype(jnp.int32)
                   for g in gs]
            idxs = [bi * _L + lane for bi in bis]
            valids = [g < jnp.float32(1.0 + 1e-6) for g in gs]
            for k in range(_K):
                plsc.addupdate_scatter(cnt_acc, [idxs[k]], ones16,
                                       mask=valids[k])
                plsc.addupdate_scatter(sum_acc, [idxs[k]], bces[k],
                                       mask=valids[k])
            return carry

        lax.fori_loop(0, _VECS // _K, inner, 0)

    start(0, 0)
    start(1, 1)

    def outer(i, carry):
        c = i * 2
        for b in range(2):
            wait(b)
            compute(b)
            nxt = c + b + 2

            @pl.when(nxt < _NCH)
            def _():
                start(nxt, b)
        return carry

    lax.fori_loop(0, _NCH // 2, outer, 0)

    cvec = zeros16
    svec = zeros16
    for v in range(_BINS):
        cs = jnp.sum(cnt_acc[pl.ds(v * _L, _L)])
        ss = jnp.sum(sum_acc[pl.ds(v * _L, _L)])
        sel = lane == v
        cvec = jnp.where(sel, cs, cvec)
        svec = jnp.where(sel, ss, svec)
    cnt_acc[pl.ds(0, _L)] = cvec
    sum_acc[pl.ds(0, _L)] = svec
    dst = pl.ds(wid * _L, _L)
    pltpu.sync_copy(cnt_acc.at[pl.ds(0, _L)], cnt_out.at[dst])
    pltpu.sync_copy(sum_acc.at[pl.ds(0, _L)], sum_out.at[dst])


def _tc_body(pred_ref, targ_ref, out_ref, acc_ref):
    step = pl.program_id(0)

    @pl.when(step == 0)
    def _init():
        for i in range(_BINS):
            acc_ref[0, i] = jnp.float32(0.0)
            acc_ref[1, i] = jnp.float32(0.0)

    p = pred_ref[...]
    t = targ_ref[...]
    a = jnp.abs(p)
    bce = jnp.maximum(p, 0.0) - p * t + jnp.log1p(jnp.exp(-a))
    g = jnp.abs(p - t)

    for i, e in enumerate(_EDGES):
        m = g < jnp.float32(e)
        acc_ref[0, i] += jnp.sum(m.astype(jnp.float32))
        acc_ref[1, i] += jnp.sum(jnp.where(m, bce, 0.0))

    @pl.when(step == _G - 1)
    def _fini():
        for i in range(_BINS):
            out_ref[0, i] = acc_ref[0, i]
            out_ref[1, i] = acc_ref[1, i]


def _fin_body(cnt_ref, sum_ref, tc_ref, out_ref):
    loss = jnp.float32(0.0)
    n = jnp.float32(0.0)
    prev_c = jnp.float32(0.0)
    prev_s = jnp.float32(0.0)
    for i in range(_BINS):
        cb = jnp.float32(0.0)
        sb = jnp.float32(0.0)
        for w in range(_NW):
            cb += cnt_ref[w, i]
            sb += sum_ref[w, i]
        cc = tc_ref[0, i]
        sc = tc_ref[1, i]
        cb += cc - prev_c
        sb += sc - prev_s
        prev_c, prev_s = cc, sc
        has = cb > 0.0
        loss += jnp.where(has, sb / jnp.maximum(cb, 1.0), 0.0)
        n += jnp.where(has, 1.0, 0.0)
    out_ref[0, 0] = jnp.where(n > 0.0, loss / jnp.maximum(n, 1.0), 0.0)


def kernel(pred, target, label_weight):
    del label_weight

    if _SC_ROWS == 0:
        cnt = jnp.zeros((_NW * _L,), jnp.float32)
        s = jnp.zeros((_NW * _L,), jnp.float32)
        return _run_tc(pred, target, cnt, s)
    pf = pred.reshape(_ROWS * _COLS)
    tf = target.reshape(_ROWS * _COLS)

    sc_call = pl.kernel(
        _sc_body,
        out_type=[
            jax.ShapeDtypeStruct((_NW * _L,), jnp.float32),
            jax.ShapeDtypeStruct((_NW * _L,), jnp.float32),
        ],
        mesh=plsc.VectorSubcoreMesh(core_axis_name="c", subcore_axis_name="s"),
        scratch_types=[
            pltpu.VMEM((_CH,), jnp.float32),
            pltpu.VMEM((_CH,), jnp.float32),
            pltpu.VMEM((_CH,), jnp.float32),
            pltpu.VMEM((_CH,), jnp.float32),
            pltpu.VMEM((_BINS * _L,), jnp.float32),
            pltpu.VMEM((_BINS * _L,), jnp.float32),
            pltpu.SemaphoreType.DMA,
            pltpu.SemaphoreType.DMA,
        ],
        compiler_params=pltpu.CompilerParams(needs_layout_passes=False),
    )
    cnt, s = sc_call(pf, tf)
    return _run_tc(pred, target, cnt, s)


def _run_tc(pred, target, cnt, s):
    tc_parts = pl.pallas_call(
        _tc_body,
        grid=(_G,),
        in_specs=[
            pl.BlockSpec((_BM, _COLS), lambda i: (i, 0)),
            pl.BlockSpec((_BM, _COLS), lambda i: (i, 0)),
        ],
        out_specs=pl.BlockSpec(memory_space=pltpu.SMEM),
        out_shape=jax.ShapeDtypeStruct((2, _BINS), jnp.float32),
        scratch_shapes=[pltpu.SMEM((2, _BINS), jnp.float32)],
        compiler_params=pltpu.CompilerParams(
            allow_input_fusion=[True, True]),
    )(pred, target)

    out = pl.pallas_call(
        _fin_body,
        in_specs=[
            pl.BlockSpec(memory_space=pltpu.SMEM),
            pl.BlockSpec(memory_space=pltpu.SMEM),
            pl.BlockSpec(memory_space=pltpu.SMEM),
        ],
        out_specs=pl.BlockSpec(memory_space=pltpu.SMEM),
        out_shape=jax.ShapeDtypeStruct((1, 1), jnp.float32),
    )(cnt.reshape(_NW, _L), s.reshape(_NW, _L), tc_parts)
    return out[0, 0]


# speedup vs baseline: 1.2936x; 1.2936x over previous
import jax
import jax.numpy as jnp
from jax import lax
from jax.experimental import pallas as pl
from jax.experimental.pallas import tpu as pltpu
from jax.experimental.pallas import tpu_sc as plsc

_BINS = 10
_ROWS, _COLS = 16384, 1000

_SC_ROWS = 0
_TC_ROWS = _ROWS - _SC_ROWS

_NSC = _SC_ROWS * _COLS
_NC, _NS, _L = 2, 16, 16
_NW = _NC * _NS
_PER_W = _NSC // _NW
_CH = 16000
_NCH = _PER_W // _CH
_VECS = _CH // _L
_K = 8

_BM = 512
_G = _TC_ROWS // _BM
_EDGES = [(i + 1) / 10.0 for i in range(_BINS - 1)] + [1.0 + 1e-6]

_P = (6.9445741e-05, 9.9626195e-01, -4.6644244e-01, 2.1866548e-01,
      -5.5459313e-02)


def _log1p_poly(z):
    c0, c1, c2, c3, c4 = (jnp.float32(c) for c in _P)
    z2 = z * z
    return (c0 + c1 * z) + z2 * ((c2 + c3 * z) + c4 * z2)


def _sc_body(pred_hbm, targ_hbm, cnt_out, sum_out,
             pbuf0, pbuf1, tbuf0, tbuf1, cnt_acc, sum_acc, sem0, sem1):
    wid = lax.axis_index("s") * _NC + lax.axis_index("c")
    base = (_ROWS - _SC_ROWS) * _COLS + wid * _PER_W
    sems = (sem0, sem1)
    lane = lax.iota(jnp.int32, _L)
    zeros16 = jnp.zeros((_L,), jnp.float32)

    for v in range(_BINS):
        cnt_acc[pl.ds(v * _L, _L)] = zeros16
        sum_acc[pl.ds(v * _L, _L)] = zeros16

    pbufs = (pbuf0, pbuf1)
    tbufs = (tbuf0, tbuf1)

    def start(c, b):
        src = pl.ds(base + c * _CH, _CH)
        pltpu.make_async_copy(pred_hbm.at[src], pbufs[b], sems[b]).start()
        pltpu.make_async_copy(targ_hbm.at[src], tbufs[b], sems[b]).start()

    def wait(b):
        dummy = pl.ds(0, _CH)
        pltpu.make_async_copy(pred_hbm.at[dummy], pbufs[b], sems[b]).wait()
        pltpu.make_async_copy(targ_hbm.at[dummy], tbufs[b], sems[b]).wait()

    ones16 = jnp.ones((_L,), jnp.float32)

    def compute(b):
        pb = pbufs[b]
        tb = tbufs[b]

        def inner(j, carry):
            base_j = j * (_K * _L)
            ps = [pb[pl.ds(base_j + k * _L, _L)] for k in range(_K)]
            ts = [tb[pl.ds(base_j + k * _L, _L)] for k in range(_K)]
            zs = [jnp.exp(-jnp.abs(p)) for p in ps]
            l1s = [_log1p_poly(z) for z in zs]
            bces = [jnp.maximum(p, 0.0) - p * t + l1
                    for p, t, l1 in zip(ps, ts, l1s)]
            gs = [jnp.abs(p - t) for p, t in zip(ps, ts)]
            bis = [(jnp.minimum(g, 0.95) * 10.0).astype(jnp.int32)
                   for g in gs]
            idxs = [bi * _L + lane for bi in bis]
            valids = [g < jnp.float32(1.0 + 1e-6) for g in gs]
            for k in range(_K):
                plsc.addupdate_scatter(cnt_acc, [idxs[k]], ones16,
                                       mask=valids[k])
                plsc.addupdate_scatter(sum_acc, [idxs[k]], bces[k],
                                       mask=valids[k])
            return carry

        lax.fori_loop(0, _VECS // _K, inner, 0)

    start(0, 0)
    start(1, 1)

    def outer(i, carry):
        c = i * 2
        for b in range(2):
            wait(b)
            compute(b)
            nxt = c + b + 2

            @pl.when(nxt < _NCH)
            def _():
                start(nxt, b)
        return carry

    lax.fori_loop(0, _NCH // 2, outer, 0)

    cvec = zeros16
    svec = zeros16
    for v in range(_BINS):
        cs = jnp.sum(cnt_acc[pl.ds(v * _L, _L)])
        ss = jnp.sum(sum_acc[pl.ds(v * _L, _L)])
        sel = lane == v
        cvec = jnp.where(sel, cs, cvec)
        svec = jnp.where(sel, ss, svec)
    cnt_acc[pl.ds(0, _L)] = cvec
    sum_acc[pl.ds(0, _L)] = svec
    dst = pl.ds(wid * _L, _L)
    pltpu.sync_copy(cnt_acc.at[pl.ds(0, _L)], cnt_out.at[dst])
    pltpu.sync_copy(sum_acc.at[pl.ds(0, _L)], sum_out.at[dst])


def _tc_body(pred_ref, targ_ref, out_ref, acc_ref):
    step = pl.program_id(0)

    @pl.when(step == 0)
    def _init():
        acc_ref[...] = jnp.zeros_like(acc_ref)

    p = pred_ref[...]
    t = targ_ref[...]
    a = jnp.abs(p)
    bce = jnp.maximum(p, 0.0) - p * t + jnp.log1p(jnp.exp(-a))
    g = jnp.abs(p - t)

    ones_l = jnp.ones((8, _BM), jnp.float32)
    for i, e in enumerate(_EDGES):
        m = g < jnp.float32(e)
        mc = m.astype(jnp.float32)
        ms = jnp.where(m, bce, 0.0)
        pc = jax.lax.dot(ones_l, mc, preferred_element_type=jnp.float32)
        psum = jax.lax.dot(ones_l, ms, preferred_element_type=jnp.float32)
        acc_ref[i] += pc
        acc_ref[_BINS + i] += psum

    @pl.when(step == _G - 1)
    def _fini():
        for i in range(_BINS):
            out_ref[0, i] = jnp.sum(acc_ref[i]) * jnp.float32(0.125)
            out_ref[1, i] = jnp.sum(acc_ref[_BINS + i]) * jnp.float32(0.125)


def _fin_body(cnt_ref, sum_ref, tc_ref, out_ref):
    loss = jnp.float32(0.0)
    n = jnp.float32(0.0)
    prev_c = jnp.float32(0.0)
    prev_s = jnp.float32(0.0)
    for i in range(_BINS):
        cb = jnp.float32(0.0)
        sb = jnp.float32(0.0)
        for w in range(_NW):
            cb += cnt_ref[w, i]
            sb += sum_ref[w, i]
        cc = tc_ref[0, i]
        sc = tc_ref[1, i]
        cb += cc - prev_c
        sb += sc - prev_s
        prev_c, prev_s = cc, sc
        has = cb > 0.0
        loss += jnp.where(has, sb / jnp.maximum(cb, 1.0), 0.0)
        n += jnp.where(has, 1.0, 0.0)
    out_ref[0, 0] = jnp.where(n > 0.0, loss / jnp.maximum(n, 1.0), 0.0)


def kernel(pred, target, label_weight):
    del label_weight

    if _SC_ROWS == 0:
        cnt = jnp.zeros((_NW * _L,), jnp.float32)
        s = jnp.zeros((_NW * _L,), jnp.float32)
        return _run_tc(pred, target, cnt, s)
    pf = pred.reshape(_ROWS * _COLS)
    tf = target.reshape(_ROWS * _COLS)

    sc_call = pl.kernel(
        _sc_body,
        out_type=[
            jax.ShapeDtypeStruct((_NW * _L,), jnp.float32),
            jax.ShapeDtypeStruct((_NW * _L,), jnp.float32),
        ],
        mesh=plsc.VectorSubcoreMesh(core_axis_name="c", subcore_axis_name="s"),
        scratch_types=[
            pltpu.VMEM((_CH,), jnp.float32),
            pltpu.VMEM((_CH,), jnp.float32),
            pltpu.VMEM((_CH,), jnp.float32),
            pltpu.VMEM((_CH,), jnp.float32),
            pltpu.VMEM((_BINS * _L,), jnp.float32),
            pltpu.VMEM((_BINS * _L,), jnp.float32),
            pltpu.SemaphoreType.DMA,
            pltpu.SemaphoreType.DMA,
        ],
        compiler_params=pltpu.CompilerParams(needs_layout_passes=False),
    )
    cnt, s = sc_call(pf, tf)
    return _run_tc(pred, target, cnt, s)


def _run_tc(pred, target, cnt, s):
    tc_parts = pl.pallas_call(
        _tc_body,
        grid=(_G,),
        in_specs=[
            pl.BlockSpec((_BM, _COLS), lambda i: (i, 0)),
            pl.BlockSpec((_BM, _COLS), lambda i: (i, 0)),
        ],
        out_specs=pl.BlockSpec(memory_space=pltpu.SMEM),
        out_shape=jax.ShapeDtypeStruct((2, _BINS), jnp.float32),
        scratch_shapes=[pltpu.VMEM((2 * _BINS, 8, _COLS), jnp.float32)],
        compiler_params=pltpu.CompilerParams(
            allow_input_fusion=[True, True]),
    )(pred, target)

    out = pl.pallas_call(
        _fin_body,
        in_specs=[
            pl.BlockSpec(memory_space=pltpu.SMEM),
            pl.BlockSpec(memory_space=pltpu.SMEM),
            pl.BlockSpec(memory_space=pltpu.SMEM),
        ],
        out_specs=pl.BlockSpec(memory_space=pltpu.SMEM),
        out_shape=jax.ShapeDtypeStruct((1, 1), jnp.float32),
    )(cnt.reshape(_NW, _L), s.reshape(_NW, _L), tc_parts)
    return out[0, 0]
